# baseline (speedup 1.0000x reference)
import jax
import jax.numpy as jnp
from jax.experimental import pallas as pl
from jax.experimental.pallas import tpu as pltpu


def _pos_embed_kernel(emb_ref, out_ref, t_ref, sems):
    i = pl.program_id(0)
    n = pl.num_programs(0)
    B, F, L = out_ref.shape
    LT = L // n
    t_ref[...] = emb_ref[...].T

    for b in range(B):
        pltpu.make_async_copy(
            t_ref, out_ref.at[b, :, pl.ds(i * LT, LT)], sems.at[b]
        ).start()

    for b in range(B):
        pltpu.make_async_copy(
            t_ref, out_ref.at[b, :, pl.ds(i * LT, LT)], sems.at[b]
        ).wait()


def kernel(x, mask, row_embed):
    B = x.shape[0]
    F = x.shape[1]
    L = x.shape[-1]
    LT = 512
    return pl.pallas_call(
        _pos_embed_kernel,
        grid=(L // LT,),
        in_specs=[pl.BlockSpec((LT, F), lambda i: (i, 0))],
        out_specs=pl.BlockSpec(memory_space=pl.ANY),
        out_shape=jax.ShapeDtypeStruct((B, F, L), jnp.float32),
        scratch_shapes=[
            pltpu.VMEM((F, LT), jnp.float32),
            pltpu.SemaphoreType.DMA((B,)),
        ],
    )(row_embed)
